# baseline (speedup 1.0000x reference)
import functools

import jax
import jax.numpy as jnp
from jax import lax
from jax.experimental import pallas as pl
from jax.experimental.pallas import tpu as pltpu
from jax.experimental.pallas import tpu_sc as plsc

N = 10000
E = 160000
NG = 128
H = 128
CUTOFF = 5.0

NWORKERS = 32
EPW = E // NWORKERS
CHUNK = 40
NCHUNK = EPW // CHUNK

BE = 2000
BN = 2000


def _swish(t):
    return t * (1.0 / (1.0 + jnp.exp(-t)))


def _sc_gather_combine(tab_a, tab_b, idx_a, idx_b, op):
    D = tab_a.shape[1]
    mesh = plsc.VectorSubcoreMesh(core_axis_name="c", subcore_axis_name="s",
                                  num_cores=2, num_subcores=16)

    def body(ta, tb, ia, ib, out, ia_v, ib_v, bufs_a, bufs_b, bufs_o,
             g_sems, w_sems):
        wid = lax.axis_index("s") * 2 + lax.axis_index("c")
        base_w = wid * EPW
        pltpu.sync_copy(ia.at[pl.ds(base_w, EPW)], ia_v)
        pltpu.sync_copy(ib.at[pl.ds(base_w, EPW)], ib_v)

        def start(ci, slot):
            isl = pl.ds(ci * CHUNK, CHUNK)
            pltpu.async_copy(ta.at[ia_v.at[isl]], bufs_a[slot], g_sems[slot])
            pltpu.async_copy(tb.at[ib_v.at[isl]], bufs_b[slot], g_sems[slot])

        def drain_g(slot):
            isl = pl.ds(0, CHUNK)
            pltpu.make_async_copy(ta.at[ia_v.at[isl]], bufs_a[slot],
                                  g_sems[slot]).wait()
            pltpu.make_async_copy(tb.at[ib_v.at[isl]], bufs_b[slot],
                                  g_sems[slot]).wait()

        def drain_w(slot):
            pltpu.make_async_copy(bufs_o[slot], out.at[pl.ds(0, CHUNK)],
                                  w_sems[slot]).wait()

        def combine(slot):
            ba, bb, bo = bufs_a[slot], bufs_b[slot], bufs_o[slot]

            @plsc.parallel_loop(0, CHUNK, step=1, unroll=4)
            def _(r):
                for c in range(D // 16):
                    sl = pl.ds(c * 16, 16)
                    a = ba[r, sl]
                    b = bb[r, sl]
                    if op == "sqdiff":
                        d = a - b
                        bo[r, sl] = d * d
                    elif op == "add":
                        bo[r, sl] = a + b
                    else:
                        bo[r, sl] = a * b

        def write(ci, slot):
            pltpu.async_copy(bufs_o[slot], out.at[pl.ds(base_w + ci * CHUNK,
                                                        CHUNK)], w_sems[slot])

        start(0, 0)

        def pair(k, carry):
            ci = 2 * k
            start(ci + 1, 1)
            drain_g(0)

            @pl.when(k > 0)
            def _():
                drain_w(0)

            combine(0)
            write(ci, 0)

            @pl.when(ci + 2 < NCHUNK)
            def _():
                start(ci + 2, 0)

            drain_g(1)

            @pl.when(k > 0)
            def _():
                drain_w(1)

            combine(1)
            write(ci + 1, 1)
            return carry

        lax.fori_loop(0, (NCHUNK - 1) // 2, pair, 0)
        drain_g(0)
        drain_w(0)
        combine(0)
        pltpu.sync_copy(bufs_o[0], out.at[pl.ds(base_w + (NCHUNK - 1) * CHUNK,
                                                CHUNK)])
        drain_w(1)

    fn = pl.kernel(
        body,
        out_type=jax.ShapeDtypeStruct((E, D), jnp.float32),
        mesh=mesh,
        scratch_types=[
            pltpu.VMEM((EPW,), jnp.int32),
            pltpu.VMEM((EPW,), jnp.int32),
            [pltpu.VMEM((CHUNK, D), jnp.float32)] * 2,
            [pltpu.VMEM((CHUNK, D), jnp.float32)] * 2,
            [pltpu.VMEM((CHUNK, D), jnp.float32)] * 2,
            [pltpu.SemaphoreType.DMA] * 2,
            [pltpu.SemaphoreType.DMA] * 2,
        ],
    )
    return fn(tab_a, tab_b, idx_a, idx_b)


NPAD = 10240


def _sc_scatter_add(m, idx, zeros_n):
    mesh = plsc.VectorSubcoreMesh(core_axis_name="c", subcore_axis_name="s",
                                  num_cores=2, num_subcores=16)
    rows = NPAD // 16

    def body(m_hbm, idx_hbm, z_hbm, out, idx_vs, bufs, agg_sh, sems):
        cid = lax.axis_index("c")
        sid = lax.axis_index("s")
        wid = sid * 2 + cid
        base_w = wid * EPW
        pltpu.sync_copy(z_hbm.at[pl.ds(sid * rows, rows)],
                        agg_sh.at[pl.ds(sid * rows, rows)])
        plsc.subcore_barrier()

        def start(ci, slot):
            base = base_w + ci * CHUNK
            pltpu.async_copy(idx_hbm.at[pl.ds(base, CHUNK)], idx_vs[slot],
                             sems[slot])
            pltpu.async_copy(m_hbm.at[pl.ds(base, CHUNK)], bufs[slot],
                             sems[slot])

        def drain(slot):
            pltpu.make_async_copy(idx_hbm.at[pl.ds(0, CHUNK)], idx_vs[slot],
                                  sems[slot]).wait()
            pltpu.make_async_copy(m_hbm.at[pl.ds(0, CHUNK)], bufs[slot],
                                  sems[slot]).wait()

        def scat(slot):
            pltpu.sync_copy(bufs[slot], agg_sh.at[idx_vs[slot]], add=True)

        start(0, 0)

        def pair(k, carry):
            ci = 2 * k
            start(ci + 1, 1)
            drain(0)
            scat(0)

            @pl.when(ci + 2 < NCHUNK)
            def _():
                start(ci + 2, 0)

            drain(1)
            scat(1)
            return carry

        lax.fori_loop(0, (NCHUNK - 1) // 2, pair, 0)
        drain(0)
        scat(0)
        plsc.subcore_barrier()
        pltpu.sync_copy(agg_sh.at[pl.ds(sid * rows, rows)],
                        out.at[cid, pl.ds(sid * rows, rows)])

    fn = pl.kernel(
        body,
        out_type=jax.ShapeDtypeStruct((2, NPAD, H), jnp.float32),
        mesh=mesh,
        scratch_types=[
            [pltpu.VMEM((CHUNK,), jnp.int32)] * 2,
            [pltpu.VMEM((CHUNK, H), jnp.float32)] * 2,
            pltpu.VMEM_SHARED((NPAD, H), jnp.float32),
            [pltpu.SemaphoreType.DMA] * 2,
        ],
    )
    return fn(m, idx, zeros_n)


def _rbf_feats(d2):
    d = jnp.sqrt(d2 + 1e-12)
    dn = d / CUTOFF
    inv = 1.0 / (dn + 1e-12)
    dn2 = dn * dn
    dn4 = dn2 * dn2
    dn5 = dn4 * dn
    env = (inv - 28.0 * dn5 + 48.0 * dn5 * dn - 21.0 * dn5 * dn2)
    env = env * (dn < 1.0).astype(jnp.float32)
    ki = lax.broadcasted_iota(jnp.int32, (1, 8), 1)
    arg = (ki.astype(jnp.float32) + 1.0) * jnp.pi * dn
    sn = jnp.sin(arg)
    mask = (ki < 6).astype(jnp.float32)
    return env * sn * mask


def _edge0_body(sq_ref, s_ref, rbfw_ref, wr_ref, wm_ref, e_ref, m_ref, d2_ref):
    d2 = jnp.sum(sq_ref[...], axis=1, keepdims=True)
    d2_ref[...] = d2
    rbf = _rbf_feats(d2)
    rbfw = rbfw_ref[...]
    w6 = jnp.dot(rbfw, wr_ref[...], preferred_element_type=jnp.float32)
    pre = s_ref[...] + jnp.dot(rbf, w6, preferred_element_type=jnp.float32)
    e = _swish(pre)
    e_ref[...] = e
    rbf_h = jnp.dot(rbf, rbfw, preferred_element_type=jnp.float32)
    m_ref[...] = _swish(jnp.dot(e, wm_ref[...],
                                preferred_element_type=jnp.float32)) * rbf_h


def _tc_edge0(sq, s, rbf8, wr, wm):
    grid = (E // BE,)
    return pl.pallas_call(
        _edge0_body,
        grid=grid,
        in_specs=[
            pl.BlockSpec((BE, H), lambda b: (b, 0)),
            pl.BlockSpec((BE, H), lambda b: (b, 0)),
            pl.BlockSpec((8, H), lambda b: (0, 0)),
            pl.BlockSpec((H, H), lambda b: (0, 0)),
            pl.BlockSpec((H, H), lambda b: (0, 0)),
        ],
        out_specs=[
            pl.BlockSpec((BE, H), lambda b: (b, 0)),
            pl.BlockSpec((BE, H), lambda b: (b, 0)),
            pl.BlockSpec((BE, 1), lambda b: (b, 0)),
        ],
        out_shape=[
            jax.ShapeDtypeStruct((E, H), jnp.float32),
            jax.ShapeDtypeStruct((E, H), jnp.float32),
            jax.ShapeDtypeStruct((E, 1), jnp.float32),
        ],
    )(sq, s, rbf8, wr, wm)


def _edge_body(d2_ref, e_ref, g_ref, rbfw_ref, wm_ref, *out_refs):
    e = e_ref[...] + _swish(g_ref[...])
    rbf_h = jnp.dot(_rbf_feats(d2_ref[...]), rbfw_ref[...],
                    preferred_element_type=jnp.float32)
    m = _swish(jnp.dot(e, wm_ref[...],
                       preferred_element_type=jnp.float32)) * rbf_h
    if len(out_refs) == 2:
        out_refs[0][...] = e
        out_refs[1][...] = m
    else:
        out_refs[0][...] = m


def _tc_edge(d2, e, g, rbf8, wm, want_e):
    grid = (E // BE,)
    eh = pl.BlockSpec((BE, H), lambda b: (b, 0))
    outs = [jax.ShapeDtypeStruct((E, H), jnp.float32)]
    out_specs = [eh]
    if want_e:
        outs = outs * 2
        out_specs = [eh, eh]
    res = pl.pallas_call(
        _edge_body,
        grid=grid,
        in_specs=[
            pl.BlockSpec((BE, 1), lambda b: (b, 0)),
            eh,
            eh,
            pl.BlockSpec((8, H), lambda b: (0, 0)),
            pl.BlockSpec((H, H), lambda b: (0, 0)),
        ],
        out_specs=out_specs,
        out_shape=outs,
    )(d2, e, g, rbf8, wm)
    return res if want_e else res[0]


def _init_body(z_ref, emb_ref, wj_ref, wi_ref, h0_ref, hjp_ref, hip_ref):
    z = z_ref[...]
    oh = (z == lax.broadcasted_iota(jnp.int32, (BN, 96), 1))
    h0 = jnp.dot(oh.astype(jnp.float32), emb_ref[...],
                 preferred_element_type=jnp.float32)
    h0_ref[...] = h0
    hjp_ref[...] = jnp.dot(h0, wj_ref[...], preferred_element_type=jnp.float32)
    hip_ref[...] = jnp.dot(h0, wi_ref[...], preferred_element_type=jnp.float32)


def _tc_init(z2, emb96, wj, wi):
    grid = (N // BN,)
    nh = pl.BlockSpec((BN, H), lambda b: (b, 0))
    return pl.pallas_call(
        _init_body,
        grid=grid,
        in_specs=[
            pl.BlockSpec((BN, 1), lambda b: (b, 0)),
            pl.BlockSpec((96, H), lambda b: (0, 0)),
            pl.BlockSpec((H, H), lambda b: (0, 0)),
            pl.BlockSpec((H, H), lambda b: (0, 0)),
        ],
        out_specs=[nh, nh, nh],
        out_shape=[jax.ShapeDtypeStruct((N, H), jnp.float32)] * 3,
    )(z2, emb96, wj, wi)


def _node_body(h_ref, agg_ref, w_ref, out_ref):
    a = agg_ref[0] + agg_ref[1]
    out_ref[...] = h_ref[...] + _swish(
        jnp.dot(a, w_ref[...], preferred_element_type=jnp.float32))


def _tc_node_update(h, agg2, w):
    grid = (N // BN,)
    nh = pl.BlockSpec((BN, H), lambda b: (b, 0))
    return pl.pallas_call(
        _node_body,
        grid=grid,
        in_specs=[
            nh,
            pl.BlockSpec((2, BN, H), lambda b: (0, b, 0)),
            pl.BlockSpec((H, H), lambda b: (0, 0)),
        ],
        out_specs=nh,
        out_shape=jax.ShapeDtypeStruct((N, H), jnp.float32),
    )(h, agg2, w)


def _out_body(h_ref, b_ref, o1_ref, o2_ref, o3_ref, w1_ref, b1_ref, w2_ref,
              b2_ref, pred_ref, ge_ref):
    k = pl.program_id(0)

    @pl.when(k == 0)
    def _():
        ge_ref[...] = jnp.zeros_like(ge_ref)

    hh = _swish(jnp.dot(h_ref[...], o1_ref[...],
                        preferred_element_type=jnp.float32))
    hh = _swish(jnp.dot(hh, o2_ref[...], preferred_element_type=jnp.float32))
    no = jnp.dot(hh, o3_ref[...], preferred_element_type=jnp.float32)
    oh = (b_ref[...] == lax.broadcasted_iota(jnp.int32, (BN, NG), 1))
    ge_ref[...] += lax.dot_general(
        oh.astype(jnp.float32), no, (((0,), (0,)), ((), ())),
        preferred_element_type=jnp.float32)

    @pl.when(k == (N // BN) - 1)
    def _():
        ge = ge_ref[...]
        hid = jnp.maximum(ge * w1_ref[...] + b1_ref[...], 0.0)
        pred_ref[...] = jnp.dot(hid, w2_ref[...],
                                preferred_element_type=jnp.float32) + b2_ref[...]


def _tc_output(h, batch2, o1, o2, o3, w1, b1, w2, b2):
    grid = (N // BN,)
    return pl.pallas_call(
        _out_body,
        grid=grid,
        in_specs=[
            pl.BlockSpec((BN, H), lambda b: (b, 0)),
            pl.BlockSpec((BN, 1), lambda b: (b, 0)),
            pl.BlockSpec((H, 256), lambda b: (0, 0)),
            pl.BlockSpec((256, 256), lambda b: (0, 0)),
            pl.BlockSpec((256, 1), lambda b: (0, 0)),
            pl.BlockSpec((1, 64), lambda b: (0, 0)),
            pl.BlockSpec((1, 64), lambda b: (0, 0)),
            pl.BlockSpec((64, 1), lambda b: (0, 0)),
            pl.BlockSpec((1, 1), lambda b: (0, 0)),
        ],
        out_specs=pl.BlockSpec((NG, 1), lambda b: (0, 0)),
        out_shape=jax.ShapeDtypeStruct((NG, 1), jnp.float32),
        scratch_shapes=[pltpu.VMEM((NG, 1), jnp.float32)],
    )(h, batch2, o1, o2, o3, w1, b1, w2, b2)


def kernel(x, pos, batch, edge_index, emb_z, lin_rbf, lin_edge, W_msg, W_upd,
           lin_out1, lin_out2, lin_out3, head_w1, head_b1, head_w2, head_b2):
    i = edge_index[0].astype(jnp.int32)
    j = edge_index[1].astype(jnp.int32)
    pos128 = jnp.pad(pos.astype(jnp.float32), ((0, 0), (0, H - 3)))
    emb96 = jnp.pad(emb_z, ((0, 1), (0, 0)))
    rbf8 = jnp.pad(lin_rbf, ((0, 2), (0, 0)))
    z2 = x.reshape(N, 1).astype(jnp.int32)
    batch2 = batch.reshape(N, 1).astype(jnp.int32)
    b1 = head_b1.reshape(1, 64)
    b2 = head_b2.reshape(1, 1)
    wj = lin_edge[0:H]
    wi = lin_edge[H:2 * H]
    wr = lin_edge[2 * H:3 * H]
    zeros_n = jnp.zeros((NPAD, H), jnp.float32)

    sq = _sc_gather_combine(pos128, pos128, i, j, "sqdiff")
    h, hjp, hip = _tc_init(z2, emb96, wj, wi)
    s = _sc_gather_combine(hjp, hip, j, i, "add")
    e, m, d2 = _tc_edge0(sq, s, rbf8, wr, W_msg[0])
    for l in range(4):
        agg2 = _sc_scatter_add(m, i, zeros_n)
        h = _tc_node_update(h, agg2, W_upd[l])
        if l < 3:
            g = _sc_gather_combine(h, h, j, i, "mul")
            if l < 2:
                e, m = _tc_edge(d2, e, g, rbf8, W_msg[l + 1], True)
            else:
                m = _tc_edge(d2, e, g, rbf8, W_msg[l + 1], False)
    return _tc_output(h, batch2, lin_out1, lin_out2, lin_out3,
                      head_w1, b1, head_w2, b2)


# speedup vs baseline: 1.0048x; 1.0048x over previous
import functools

import jax
import jax.numpy as jnp
from jax import lax
from jax.experimental import pallas as pl
from jax.experimental.pallas import tpu as pltpu
from jax.experimental.pallas import tpu_sc as plsc

N = 10000
E = 160000
NG = 128
H = 128
CUTOFF = 5.0

NWORKERS = 32
EPW = E // NWORKERS
CHUNK = 40
NCHUNK = EPW // CHUNK

BE = 2000
BN = 2000


def _swish(t):
    return t * (1.0 / (1.0 + jnp.exp(-t)))


def _sc_gather_combine(tab_a, tab_b, idx_a, idx_b, op, out_w=None):
    D = tab_a.shape[1]
    OW = D if out_w is None else out_w
    mesh = plsc.VectorSubcoreMesh(core_axis_name="c", subcore_axis_name="s",
                                  num_cores=2, num_subcores=16)

    def body(ta, tb, ia, ib, out, ia_v, ib_v, bufs_a, bufs_b, bufs_o,
             g_sems, w_sems):
        wid = lax.axis_index("s") * 2 + lax.axis_index("c")
        base_w = wid * EPW
        pltpu.sync_copy(ia.at[pl.ds(base_w, EPW)], ia_v)
        pltpu.sync_copy(ib.at[pl.ds(base_w, EPW)], ib_v)

        def start(ci, slot):
            isl = pl.ds(ci * CHUNK, CHUNK)
            pltpu.async_copy(ta.at[ia_v.at[isl]], bufs_a[slot], g_sems[slot])
            pltpu.async_copy(tb.at[ib_v.at[isl]], bufs_b[slot], g_sems[slot])

        def drain_g(slot):
            isl = pl.ds(0, CHUNK)
            pltpu.make_async_copy(ta.at[ia_v.at[isl]], bufs_a[slot],
                                  g_sems[slot]).wait()
            pltpu.make_async_copy(tb.at[ib_v.at[isl]], bufs_b[slot],
                                  g_sems[slot]).wait()

        def drain_w(slot):
            pltpu.make_async_copy(bufs_o[slot], out.at[pl.ds(0, CHUNK)],
                                  w_sems[slot]).wait()

        def combine(slot):
            ba, bb, bo = bufs_a[slot], bufs_b[slot], bufs_o[slot]

            @plsc.parallel_loop(0, CHUNK, step=1, unroll=4)
            def _(r):
                for c in range(OW // 16):
                    sl = pl.ds(c * 16, 16)
                    a = ba[r, sl]
                    b = bb[r, sl]
                    if op == "sqdiff":
                        d = a - b
                        bo[r, sl] = d * d
                    elif op == "add":
                        bo[r, sl] = a + b
                    else:
                        bo[r, sl] = a * b

        def write(ci, slot):
            pltpu.async_copy(bufs_o[slot], out.at[pl.ds(base_w + ci * CHUNK,
                                                        CHUNK)], w_sems[slot])

        start(0, 0)

        def pair(k, carry):
            ci = 2 * k
            start(ci + 1, 1)
            drain_g(0)

            @pl.when(k > 0)
            def _():
                drain_w(0)

            combine(0)
            write(ci, 0)

            @pl.when(ci + 2 < NCHUNK)
            def _():
                start(ci + 2, 0)

            drain_g(1)

            @pl.when(k > 0)
            def _():
                drain_w(1)

            combine(1)
            write(ci + 1, 1)
            return carry

        lax.fori_loop(0, (NCHUNK - 1) // 2, pair, 0)
        drain_g(0)
        drain_w(0)
        combine(0)
        pltpu.sync_copy(bufs_o[0], out.at[pl.ds(base_w + (NCHUNK - 1) * CHUNK,
                                                CHUNK)])
        drain_w(1)

    fn = pl.kernel(
        body,
        out_type=jax.ShapeDtypeStruct((E, OW), jnp.float32),
        mesh=mesh,
        scratch_types=[
            pltpu.VMEM((EPW,), jnp.int32),
            pltpu.VMEM((EPW,), jnp.int32),
            [pltpu.VMEM((CHUNK, D), jnp.float32)] * 2,
            [pltpu.VMEM((CHUNK, D), jnp.float32)] * 2,
            [pltpu.VMEM((CHUNK, OW), jnp.float32)] * 2,
            [pltpu.SemaphoreType.DMA] * 2,
            [pltpu.SemaphoreType.DMA] * 2,
        ],
    )
    return fn(tab_a, tab_b, idx_a, idx_b)


NPAD = 10240


def _sc_scatter_add(m, idx, zeros_n):
    mesh = plsc.VectorSubcoreMesh(core_axis_name="c", subcore_axis_name="s",
                                  num_cores=2, num_subcores=16)
    rows = NPAD // 16

    def body(m_hbm, idx_hbm, z_hbm, out, idx_vs, bufs, agg_sh, sems):
        cid = lax.axis_index("c")
        sid = lax.axis_index("s")
        wid = sid * 2 + cid
        base_w = wid * EPW
        pltpu.sync_copy(z_hbm.at[pl.ds(sid * rows, rows)],
                        agg_sh.at[pl.ds(sid * rows, rows)])
        plsc.subcore_barrier()

        def start(ci, slot):
            base = base_w + ci * CHUNK
            pltpu.async_copy(idx_hbm.at[pl.ds(base, CHUNK)], idx_vs[slot],
                             sems[slot])
            pltpu.async_copy(m_hbm.at[pl.ds(base, CHUNK)], bufs[slot],
                             sems[slot])

        def drain(slot):
            pltpu.make_async_copy(idx_hbm.at[pl.ds(0, CHUNK)], idx_vs[slot],
                                  sems[slot]).wait()
            pltpu.make_async_copy(m_hbm.at[pl.ds(0, CHUNK)], bufs[slot],
                                  sems[slot]).wait()

        def scat(slot):
            pltpu.sync_copy(bufs[slot], agg_sh.at[idx_vs[slot]], add=True)

        start(0, 0)

        def pair(k, carry):
            ci = 2 * k
            start(ci + 1, 1)
            drain(0)
            scat(0)

            @pl.when(ci + 2 < NCHUNK)
            def _():
                start(ci + 2, 0)

            drain(1)
            scat(1)
            return carry

        lax.fori_loop(0, (NCHUNK - 1) // 2, pair, 0)
        drain(0)
        scat(0)
        plsc.subcore_barrier()
        pltpu.sync_copy(agg_sh.at[pl.ds(sid * rows, rows)],
                        out.at[cid, pl.ds(sid * rows, rows)])

    fn = pl.kernel(
        body,
        out_type=jax.ShapeDtypeStruct((2, NPAD, H), jnp.float32),
        mesh=mesh,
        scratch_types=[
            [pltpu.VMEM((CHUNK,), jnp.int32)] * 2,
            [pltpu.VMEM((CHUNK, H), jnp.float32)] * 2,
            pltpu.VMEM_SHARED((NPAD, H), jnp.float32),
            [pltpu.SemaphoreType.DMA] * 2,
        ],
    )
    return fn(m, idx, zeros_n)


def _rbf_feats(d2):
    d = jnp.sqrt(d2 + 1e-12)
    dn = d / CUTOFF
    inv = 1.0 / (dn + 1e-12)
    dn2 = dn * dn
    dn4 = dn2 * dn2
    dn5 = dn4 * dn
    env = (inv - 28.0 * dn5 + 48.0 * dn5 * dn - 21.0 * dn5 * dn2)
    env = env * (dn < 1.0).astype(jnp.float32)
    ki = lax.broadcasted_iota(jnp.int32, (1, 8), 1)
    arg = (ki.astype(jnp.float32) + 1.0) * jnp.pi * dn
    sn = jnp.sin(arg)
    mask = (ki < 6).astype(jnp.float32)
    return env * sn * mask


def _edge0_body(sq_ref, s_ref, rbfw_ref, wr_ref, wm_ref, e_ref, m_ref, d2_ref):
    d2 = jnp.sum(sq_ref[...], axis=1, keepdims=True)
    d2_ref[...] = d2
    rbf = _rbf_feats(d2)
    rbfw = rbfw_ref[...]
    w6 = jnp.dot(rbfw, wr_ref[...], preferred_element_type=jnp.float32)
    pre = s_ref[...] + jnp.dot(rbf, w6, preferred_element_type=jnp.float32)
    e = _swish(pre)
    e_ref[...] = e
    rbf_h = jnp.dot(rbf, rbfw, preferred_element_type=jnp.float32)
    m_ref[...] = _swish(jnp.dot(e, wm_ref[...],
                                preferred_element_type=jnp.float32)) * rbf_h


def _tc_edge0(sq, s, rbf8, wr, wm):
    grid = (E // BE,)
    return pl.pallas_call(
        _edge0_body,
        grid=grid,
        in_specs=[
            pl.BlockSpec((BE, 16), lambda b: (b, 0)),
            pl.BlockSpec((BE, H), lambda b: (b, 0)),
            pl.BlockSpec((8, H), lambda b: (0, 0)),
            pl.BlockSpec((H, H), lambda b: (0, 0)),
            pl.BlockSpec((H, H), lambda b: (0, 0)),
        ],
        out_specs=[
            pl.BlockSpec((BE, H), lambda b: (b, 0)),
            pl.BlockSpec((BE, H), lambda b: (b, 0)),
            pl.BlockSpec((BE, 1), lambda b: (b, 0)),
        ],
        out_shape=[
            jax.ShapeDtypeStruct((E, H), jnp.float32),
            jax.ShapeDtypeStruct((E, H), jnp.float32),
            jax.ShapeDtypeStruct((E, 1), jnp.float32),
        ],
    )(sq, s, rbf8, wr, wm)


def _edge_body(d2_ref, e_ref, g_ref, rbfw_ref, wm_ref, *out_refs):
    e = e_ref[...] + _swish(g_ref[...])
    rbf_h = jnp.dot(_rbf_feats(d2_ref[...]), rbfw_ref[...],
                    preferred_element_type=jnp.float32)
    m = _swish(jnp.dot(e, wm_ref[...],
                       preferred_element_type=jnp.float32)) * rbf_h
    if len(out_refs) == 2:
        out_refs[0][...] = e
        out_refs[1][...] = m
    else:
        out_refs[0][...] = m


def _tc_edge(d2, e, g, rbf8, wm, want_e):
    grid = (E // BE,)
    eh = pl.BlockSpec((BE, H), lambda b: (b, 0))
    outs = [jax.ShapeDtypeStruct((E, H), jnp.float32)]
    out_specs = [eh]
    if want_e:
        outs = outs * 2
        out_specs = [eh, eh]
    res = pl.pallas_call(
        _edge_body,
        grid=grid,
        in_specs=[
            pl.BlockSpec((BE, 1), lambda b: (b, 0)),
            eh,
            eh,
            pl.BlockSpec((8, H), lambda b: (0, 0)),
            pl.BlockSpec((H, H), lambda b: (0, 0)),
        ],
        out_specs=out_specs,
        out_shape=outs,
    )(d2, e, g, rbf8, wm)
    return res if want_e else res[0]


def _init_body(z_ref, emb_ref, wj_ref, wi_ref, h0_ref, hjp_ref, hip_ref):
    z = z_ref[...]
    oh = (z == lax.broadcasted_iota(jnp.int32, (BN, 96), 1))
    h0 = jnp.dot(oh.astype(jnp.float32), emb_ref[...],
                 preferred_element_type=jnp.float32)
    h0_ref[...] = h0
    hjp_ref[...] = jnp.dot(h0, wj_ref[...], preferred_element_type=jnp.float32)
    hip_ref[...] = jnp.dot(h0, wi_ref[...], preferred_element_type=jnp.float32)


def _tc_init(z2, emb96, wj, wi):
    grid = (N // BN,)
    nh = pl.BlockSpec((BN, H), lambda b: (b, 0))
    return pl.pallas_call(
        _init_body,
        grid=grid,
        in_specs=[
            pl.BlockSpec((BN, 1), lambda b: (b, 0)),
            pl.BlockSpec((96, H), lambda b: (0, 0)),
            pl.BlockSpec((H, H), lambda b: (0, 0)),
            pl.BlockSpec((H, H), lambda b: (0, 0)),
        ],
        out_specs=[nh, nh, nh],
        out_shape=[jax.ShapeDtypeStruct((N, H), jnp.float32)] * 3,
    )(z2, emb96, wj, wi)


def _node_body(h_ref, agg_ref, w_ref, out_ref):
    a = agg_ref[0] + agg_ref[1]
    out_ref[...] = h_ref[...] + _swish(
        jnp.dot(a, w_ref[...], preferred_element_type=jnp.float32))


def _tc_node_update(h, agg2, w):
    grid = (N // BN,)
    nh = pl.BlockSpec((BN, H), lambda b: (b, 0))
    return pl.pallas_call(
        _node_body,
        grid=grid,
        in_specs=[
            nh,
            pl.BlockSpec((2, BN, H), lambda b: (0, b, 0)),
            pl.BlockSpec((H, H), lambda b: (0, 0)),
        ],
        out_specs=nh,
        out_shape=jax.ShapeDtypeStruct((N, H), jnp.float32),
    )(h, agg2, w)


def _out_body(h_ref, b_ref, o1_ref, o2_ref, o3_ref, w1_ref, b1_ref, w2_ref,
              b2_ref, pred_ref, ge_ref):
    k = pl.program_id(0)

    @pl.when(k == 0)
    def _():
        ge_ref[...] = jnp.zeros_like(ge_ref)

    hh = _swish(jnp.dot(h_ref[...], o1_ref[...],
                        preferred_element_type=jnp.float32))
    hh = _swish(jnp.dot(hh, o2_ref[...], preferred_element_type=jnp.float32))
    no = jnp.dot(hh, o3_ref[...], preferred_element_type=jnp.float32)
    oh = (b_ref[...] == lax.broadcasted_iota(jnp.int32, (BN, NG), 1))
    ge_ref[...] += lax.dot_general(
        oh.astype(jnp.float32), no, (((0,), (0,)), ((), ())),
        preferred_element_type=jnp.float32)

    @pl.when(k == (N // BN) - 1)
    def _():
        ge = ge_ref[...]
        hid = jnp.maximum(ge * w1_ref[...] + b1_ref[...], 0.0)
        pred_ref[...] = jnp.dot(hid, w2_ref[...],
                                preferred_element_type=jnp.float32) + b2_ref[...]


def _tc_output(h, batch2, o1, o2, o3, w1, b1, w2, b2):
    grid = (N // BN,)
    return pl.pallas_call(
        _out_body,
        grid=grid,
        in_specs=[
            pl.BlockSpec((BN, H), lambda b: (b, 0)),
            pl.BlockSpec((BN, 1), lambda b: (b, 0)),
            pl.BlockSpec((H, 256), lambda b: (0, 0)),
            pl.BlockSpec((256, 256), lambda b: (0, 0)),
            pl.BlockSpec((256, 1), lambda b: (0, 0)),
            pl.BlockSpec((1, 64), lambda b: (0, 0)),
            pl.BlockSpec((1, 64), lambda b: (0, 0)),
            pl.BlockSpec((64, 1), lambda b: (0, 0)),
            pl.BlockSpec((1, 1), lambda b: (0, 0)),
        ],
        out_specs=pl.BlockSpec((NG, 1), lambda b: (0, 0)),
        out_shape=jax.ShapeDtypeStruct((NG, 1), jnp.float32),
        scratch_shapes=[pltpu.VMEM((NG, 1), jnp.float32)],
    )(h, batch2, o1, o2, o3, w1, b1, w2, b2)


def kernel(x, pos, batch, edge_index, emb_z, lin_rbf, lin_edge, W_msg, W_upd,
           lin_out1, lin_out2, lin_out3, head_w1, head_b1, head_w2, head_b2):
    i = edge_index[0].astype(jnp.int32)
    j = edge_index[1].astype(jnp.int32)
    pos128 = jnp.pad(pos.astype(jnp.float32), ((0, 0), (0, H - 3)))
    emb96 = jnp.pad(emb_z, ((0, 1), (0, 0)))
    rbf8 = jnp.pad(lin_rbf, ((0, 2), (0, 0)))
    z2 = x.reshape(N, 1).astype(jnp.int32)
    batch2 = batch.reshape(N, 1).astype(jnp.int32)
    b1 = head_b1.reshape(1, 64)
    b2 = head_b2.reshape(1, 1)
    wj = lin_edge[0:H]
    wi = lin_edge[H:2 * H]
    wr = lin_edge[2 * H:3 * H]
    zeros_n = jnp.zeros((NPAD, H), jnp.float32)

    sq = _sc_gather_combine(pos128, pos128, i, j, "sqdiff", out_w=16)
    h, hjp, hip = _tc_init(z2, emb96, wj, wi)
    s = _sc_gather_combine(hjp, hip, j, i, "add")
    e, m, d2 = _tc_edge0(sq, s, rbf8, wr, W_msg[0])
    for l in range(4):
        agg2 = _sc_scatter_add(m, i, zeros_n)
        h = _tc_node_update(h, agg2, W_upd[l])
        if l < 3:
            g = _sc_gather_combine(h, h, j, i, "mul")
            if l < 2:
                e, m = _tc_edge(d2, e, g, rbf8, W_msg[l + 1], True)
            else:
                m = _tc_edge(d2, e, g, rbf8, W_msg[l + 1], False)
    return _tc_output(h, batch2, lin_out1, lin_out2, lin_out3,
                      head_w1, b1, head_w2, b2)


# speedup vs baseline: 1.4177x; 1.4109x over previous
import functools

import jax
import jax.numpy as jnp
from jax import lax
from jax.experimental import pallas as pl
from jax.experimental.pallas import tpu as pltpu
from jax.experimental.pallas import tpu_sc as plsc

N = 10000
E = 160000
NG = 128
H = 128
CUTOFF = 5.0

NWORKERS = 32
EPW = E // NWORKERS
CHUNK = 40
NCHUNK = EPW // CHUNK

BE = 2000
BN = 2000


def _swish(t):
    return t * (1.0 / (1.0 + jnp.exp(-t)))


def _sc_gather_combine(tab_a, tab_b, idx_a, idx_b, op, out_w=None):
    D = tab_a.shape[1]
    OW = D if out_w is None else out_w
    mesh = plsc.VectorSubcoreMesh(core_axis_name="c", subcore_axis_name="s",
                                  num_cores=2, num_subcores=16)

    def body(ta, tb, ia, ib, out, ia_v, ib_v, bufs_a, bufs_b, bufs_o,
             g_sems, w_sems):
        wid = lax.axis_index("s") * 2 + lax.axis_index("c")
        base_w = wid * EPW
        pltpu.sync_copy(ia.at[pl.ds(base_w, EPW)], ia_v)
        pltpu.sync_copy(ib.at[pl.ds(base_w, EPW)], ib_v)

        def start(ci, slot):
            isl = pl.ds(ci * CHUNK, CHUNK)
            pltpu.async_copy(ta.at[ia_v.at[isl]], bufs_a[slot], g_sems[slot])
            pltpu.async_copy(tb.at[ib_v.at[isl]], bufs_b[slot], g_sems[slot])

        def drain_g(slot):
            isl = pl.ds(0, CHUNK)
            pltpu.make_async_copy(ta.at[ia_v.at[isl]], bufs_a[slot],
                                  g_sems[slot]).wait()
            pltpu.make_async_copy(tb.at[ib_v.at[isl]], bufs_b[slot],
                                  g_sems[slot]).wait()

        def drain_w(slot):
            pltpu.make_async_copy(bufs_o[slot], out.at[pl.ds(0, CHUNK)],
                                  w_sems[slot]).wait()

        def combine(slot):
            ba, bb, bo = bufs_a[slot], bufs_b[slot], bufs_o[slot]

            @plsc.parallel_loop(0, CHUNK, step=1, unroll=4)
            def _(r):
                for c in range(OW // 16):
                    sl = pl.ds(c * 16, 16)
                    a = ba[r, sl]
                    b = bb[r, sl]
                    if op == "sqdiff":
                        d = a - b
                        bo[r, sl] = d * d
                    elif op == "add":
                        bo[r, sl] = a + b
                    else:
                        bo[r, sl] = a * b

        def write(ci, slot):
            pltpu.async_copy(bufs_o[slot], out.at[pl.ds(base_w + ci * CHUNK,
                                                        CHUNK)], w_sems[slot])

        start(0, 0)

        def pair(k, carry):
            ci = 2 * k
            start(ci + 1, 1)
            drain_g(0)

            @pl.when(k > 0)
            def _():
                drain_w(0)

            combine(0)
            write(ci, 0)

            @pl.when(ci + 2 < NCHUNK)
            def _():
                start(ci + 2, 0)

            drain_g(1)

            @pl.when(k > 0)
            def _():
                drain_w(1)

            combine(1)
            write(ci + 1, 1)
            return carry

        lax.fori_loop(0, (NCHUNK - 1) // 2, pair, 0)
        drain_g(0)
        drain_w(0)
        combine(0)
        pltpu.sync_copy(bufs_o[0], out.at[pl.ds(base_w + (NCHUNK - 1) * CHUNK,
                                                CHUNK)])
        drain_w(1)

    fn = pl.kernel(
        body,
        out_type=jax.ShapeDtypeStruct((E, OW), jnp.float32),
        mesh=mesh,
        scratch_types=[
            pltpu.VMEM((EPW,), jnp.int32),
            pltpu.VMEM((EPW,), jnp.int32),
            [pltpu.VMEM((CHUNK, D), jnp.float32)] * 2,
            [pltpu.VMEM((CHUNK, D), jnp.float32)] * 2,
            [pltpu.VMEM((CHUNK, OW), jnp.float32)] * 2,
            [pltpu.SemaphoreType.DMA] * 2,
            [pltpu.SemaphoreType.DMA] * 2,
        ],
    )
    return fn(tab_a, tab_b, idx_a, idx_b)


NPAD = 10240


def _sc_scatter_add(m, idx, zeros_n):
    mesh = plsc.VectorSubcoreMesh(core_axis_name="c", subcore_axis_name="s",
                                  num_cores=2, num_subcores=16)
    rows = NPAD // 16

    def body(m_hbm, idx_hbm, z_hbm, out, idx_vs, bufs, agg_sh, sems):
        cid = lax.axis_index("c")
        sid = lax.axis_index("s")
        wid = sid * 2 + cid
        base_w = wid * EPW
        pltpu.sync_copy(z_hbm.at[pl.ds(sid * rows, rows)],
                        agg_sh.at[pl.ds(sid * rows, rows)])
        plsc.subcore_barrier()

        def start(ci, slot):
            base = base_w + ci * CHUNK
            pltpu.async_copy(idx_hbm.at[pl.ds(base, CHUNK)], idx_vs[slot],
                             sems[slot])
            pltpu.async_copy(m_hbm.at[pl.ds(base, CHUNK)], bufs[slot],
                             sems[slot])

        def drain(slot):
            pltpu.make_async_copy(idx_hbm.at[pl.ds(0, CHUNK)], idx_vs[slot],
                                  sems[slot]).wait()
            pltpu.make_async_copy(m_hbm.at[pl.ds(0, CHUNK)], bufs[slot],
                                  sems[slot]).wait()

        def scat(slot):
            pltpu.sync_copy(bufs[slot], agg_sh.at[idx_vs[slot]], add=True)

        start(0, 0)

        def pair(k, carry):
            ci = 2 * k
            start(ci + 1, 1)
            drain(0)
            scat(0)

            @pl.when(ci + 2 < NCHUNK)
            def _():
                start(ci + 2, 0)

            drain(1)
            scat(1)
            return carry

        lax.fori_loop(0, (NCHUNK - 1) // 2, pair, 0)
        drain(0)
        scat(0)
        plsc.subcore_barrier()
        pltpu.sync_copy(agg_sh.at[pl.ds(sid * rows, rows)],
                        out.at[cid, pl.ds(sid * rows, rows)])

    fn = pl.kernel(
        body,
        out_type=jax.ShapeDtypeStruct((2, NPAD, H), jnp.float32),
        mesh=mesh,
        scratch_types=[
            [pltpu.VMEM((CHUNK,), jnp.int32)] * 2,
            [pltpu.VMEM((CHUNK, H), jnp.float32)] * 2,
            pltpu.VMEM_SHARED((NPAD, H), jnp.float32),
            [pltpu.SemaphoreType.DMA] * 2,
        ],
    )
    return fn(m, idx, zeros_n)


def _rbf_feats(d2):
    d = jnp.sqrt(d2 + 1e-12)
    dn = d / CUTOFF
    inv = 1.0 / (dn + 1e-12)
    dn2 = dn * dn
    dn4 = dn2 * dn2
    dn5 = dn4 * dn
    env = (inv - 28.0 * dn5 + 48.0 * dn5 * dn - 21.0 * dn5 * dn2)
    env = env * (dn < 1.0).astype(jnp.float32)
    ki = lax.broadcasted_iota(jnp.int32, (1, 8), 1)
    arg = (ki.astype(jnp.float32) + 1.0) * jnp.pi * dn
    sn = jnp.sin(arg)
    mask = (ki < 6).astype(jnp.float32)
    return env * sn * mask


def _edge0_body(sq_ref, s_ref, rbfw_ref, wr_ref, wm_ref, e_ref, m_ref,
                rbfh_ref):
    d2 = jnp.sum(sq_ref[...], axis=1, keepdims=True)
    rbf = _rbf_feats(d2)
    rbfw = rbfw_ref[...]
    w6 = jnp.dot(rbfw, wr_ref[...], preferred_element_type=jnp.float32)
    pre = s_ref[...] + jnp.dot(rbf, w6, preferred_element_type=jnp.float32)
    e = _swish(pre)
    e_ref[...] = e
    rbf_h = jnp.dot(rbf, rbfw, preferred_element_type=jnp.float32)
    rbfh_ref[...] = rbf_h
    m_ref[...] = _swish(jnp.dot(e, wm_ref[...],
                                preferred_element_type=jnp.float32)) * rbf_h


def _tc_edge0(sq, s, rbf8, wr, wm):
    grid = (E // BE,)
    return pl.pallas_call(
        _edge0_body,
        grid=grid,
        in_specs=[
            pl.BlockSpec((BE, 16), lambda b: (b, 0)),
            pl.BlockSpec((BE, H), lambda b: (b, 0)),
            pl.BlockSpec((8, H), lambda b: (0, 0)),
            pl.BlockSpec((H, H), lambda b: (0, 0)),
            pl.BlockSpec((H, H), lambda b: (0, 0)),
        ],
        out_specs=[
            pl.BlockSpec((BE, H), lambda b: (b, 0)),
            pl.BlockSpec((BE, H), lambda b: (b, 0)),
            pl.BlockSpec((BE, H), lambda b: (b, 0)),
        ],
        out_shape=[
            jax.ShapeDtypeStruct((E, H), jnp.float32),
            jax.ShapeDtypeStruct((E, H), jnp.float32),
            jax.ShapeDtypeStruct((E, H), jnp.float32),
        ],
    )(sq, s, rbf8, wr, wm)


def _edge_body(rbfh_ref, e_ref, g_ref, wm_ref, *out_refs):
    e = e_ref[...] + _swish(g_ref[...])
    m = _swish(jnp.dot(e, wm_ref[...],
                       preferred_element_type=jnp.float32)) * rbfh_ref[...]
    if len(out_refs) == 2:
        out_refs[0][...] = e
        out_refs[1][...] = m
    else:
        out_refs[0][...] = m


def _tc_edge(rbfh, e, g, wm, want_e):
    grid = (E // BE,)
    eh = pl.BlockSpec((BE, H), lambda b: (b, 0))
    outs = [jax.ShapeDtypeStruct((E, H), jnp.float32)]
    out_specs = [eh]
    if want_e:
        outs = outs * 2
        out_specs = [eh, eh]
    res = pl.pallas_call(
        _edge_body,
        grid=grid,
        in_specs=[
            eh,
            eh,
            eh,
            pl.BlockSpec((H, H), lambda b: (0, 0)),
        ],
        out_specs=out_specs,
        out_shape=outs,
    )(rbfh, e, g, wm)
    return res if want_e else res[0]


def _init_body(z_ref, emb_ref, wj_ref, wi_ref, h0_ref, hjp_ref, hip_ref):
    z = z_ref[...]
    oh = (z == lax.broadcasted_iota(jnp.int32, (BN, 96), 1))
    h0 = jnp.dot(oh.astype(jnp.float32), emb_ref[...],
                 preferred_element_type=jnp.float32)
    h0_ref[...] = h0
    hjp_ref[...] = jnp.dot(h0, wj_ref[...], preferred_element_type=jnp.float32)
    hip_ref[...] = jnp.dot(h0, wi_ref[...], preferred_element_type=jnp.float32)


def _tc_init(z2, emb96, wj, wi):
    grid = (N // BN,)
    nh = pl.BlockSpec((BN, H), lambda b: (b, 0))
    return pl.pallas_call(
        _init_body,
        grid=grid,
        in_specs=[
            pl.BlockSpec((BN, 1), lambda b: (b, 0)),
            pl.BlockSpec((96, H), lambda b: (0, 0)),
            pl.BlockSpec((H, H), lambda b: (0, 0)),
            pl.BlockSpec((H, H), lambda b: (0, 0)),
        ],
        out_specs=[nh, nh, nh],
        out_shape=[jax.ShapeDtypeStruct((N, H), jnp.float32)] * 3,
    )(z2, emb96, wj, wi)


def _node_body(h_ref, agg_ref, w_ref, out_ref):
    a = agg_ref[0] + agg_ref[1]
    out_ref[...] = h_ref[...] + _swish(
        jnp.dot(a, w_ref[...], preferred_element_type=jnp.float32))


def _tc_node_update(h, agg2, w):
    grid = (N // BN,)
    nh = pl.BlockSpec((BN, H), lambda b: (b, 0))
    return pl.pallas_call(
        _node_body,
        grid=grid,
        in_specs=[
            nh,
            pl.BlockSpec((2, BN, H), lambda b: (0, b, 0)),
            pl.BlockSpec((H, H), lambda b: (0, 0)),
        ],
        out_specs=nh,
        out_shape=jax.ShapeDtypeStruct((N, H), jnp.float32),
    )(h, agg2, w)


def _out_body(h_ref, b_ref, o1_ref, o2_ref, o3_ref, w1_ref, b1_ref, w2_ref,
              b2_ref, pred_ref, ge_ref):
    k = pl.program_id(0)

    @pl.when(k == 0)
    def _():
        ge_ref[...] = jnp.zeros_like(ge_ref)

    hh = _swish(jnp.dot(h_ref[...], o1_ref[...],
                        preferred_element_type=jnp.float32))
    hh = _swish(jnp.dot(hh, o2_ref[...], preferred_element_type=jnp.float32))
    no = jnp.dot(hh, o3_ref[...], preferred_element_type=jnp.float32)
    oh = (b_ref[...] == lax.broadcasted_iota(jnp.int32, (BN, NG), 1))
    ge_ref[...] += lax.dot_general(
        oh.astype(jnp.float32), no, (((0,), (0,)), ((), ())),
        preferred_element_type=jnp.float32)

    @pl.when(k == (N // BN) - 1)
    def _():
        ge = ge_ref[...]
        hid = jnp.maximum(ge * w1_ref[...] + b1_ref[...], 0.0)
        pred_ref[...] = jnp.dot(hid, w2_ref[...],
                                preferred_element_type=jnp.float32) + b2_ref[...]


def _tc_output(h, batch2, o1, o2, o3, w1, b1, w2, b2):
    grid = (N // BN,)
    return pl.pallas_call(
        _out_body,
        grid=grid,
        in_specs=[
            pl.BlockSpec((BN, H), lambda b: (b, 0)),
            pl.BlockSpec((BN, 1), lambda b: (b, 0)),
            pl.BlockSpec((H, 256), lambda b: (0, 0)),
            pl.BlockSpec((256, 256), lambda b: (0, 0)),
            pl.BlockSpec((256, 1), lambda b: (0, 0)),
            pl.BlockSpec((1, 64), lambda b: (0, 0)),
            pl.BlockSpec((1, 64), lambda b: (0, 0)),
            pl.BlockSpec((64, 1), lambda b: (0, 0)),
            pl.BlockSpec((1, 1), lambda b: (0, 0)),
        ],
        out_specs=pl.BlockSpec((NG, 1), lambda b: (0, 0)),
        out_shape=jax.ShapeDtypeStruct((NG, 1), jnp.float32),
        scratch_shapes=[pltpu.VMEM((NG, 1), jnp.float32)],
    )(h, batch2, o1, o2, o3, w1, b1, w2, b2)


def kernel(x, pos, batch, edge_index, emb_z, lin_rbf, lin_edge, W_msg, W_upd,
           lin_out1, lin_out2, lin_out3, head_w1, head_b1, head_w2, head_b2):
    i = edge_index[0].astype(jnp.int32)
    j = edge_index[1].astype(jnp.int32)
    pos128 = jnp.pad(pos.astype(jnp.float32), ((0, 0), (0, H - 3)))
    emb96 = jnp.pad(emb_z, ((0, 1), (0, 0)))
    rbf8 = jnp.pad(lin_rbf, ((0, 2), (0, 0)))
    z2 = x.reshape(N, 1).astype(jnp.int32)
    batch2 = batch.reshape(N, 1).astype(jnp.int32)
    b1 = head_b1.reshape(1, 64)
    b2 = head_b2.reshape(1, 1)
    wj = lin_edge[0:H]
    wi = lin_edge[H:2 * H]
    wr = lin_edge[2 * H:3 * H]
    zeros_n = jnp.zeros((NPAD, H), jnp.float32)

    sq = _sc_gather_combine(pos128, pos128, i, j, "sqdiff", out_w=16)
    h, hjp, hip = _tc_init(z2, emb96, wj, wi)
    s = _sc_gather_combine(hjp, hip, j, i, "add")
    e, m, rbfh = _tc_edge0(sq, s, rbf8, wr, W_msg[0])
    for l in range(4):
        agg2 = _sc_scatter_add(m, i, zeros_n)
        h = _tc_node_update(h, agg2, W_upd[l])
        if l < 3:
            g = _sc_gather_combine(h, h, j, i, "mul")
            if l < 2:
                e, m = _tc_edge(rbfh, e, g, W_msg[l + 1], True)
            else:
                m = _tc_edge(rbfh, e, g, W_msg[l + 1], False)
    return _tc_output(h, batch2, lin_out1, lin_out2, lin_out3,
                      head_w1, b1, head_w2, b2)


# speedup vs baseline: 1.4550x; 1.0263x over previous
import functools

import jax
import jax.numpy as jnp
from jax import lax
from jax.experimental import pallas as pl
from jax.experimental.pallas import tpu as pltpu
from jax.experimental.pallas import tpu_sc as plsc

N = 10000
E = 160000
NG = 128
H = 128
CUTOFF = 5.0

NWORKERS = 32
EPW = E // NWORKERS
CHUNK = 40
NCHUNK = EPW // CHUNK

BE = 4000
BN = 2000


def _swish(t):
    return t * (1.0 / (1.0 + jnp.exp(-t)))


def _sc_gather_combine(tab_a, tab_b, idx_a, idx_b, op, out_w=None):
    D = tab_a.shape[1]
    OW = D if out_w is None else out_w
    mesh = plsc.VectorSubcoreMesh(core_axis_name="c", subcore_axis_name="s",
                                  num_cores=2, num_subcores=16)

    def body(ta, tb, ia, ib, out, ia_v, ib_v, bufs_a, bufs_b, bufs_o,
             g_sems, w_sems):
        wid = lax.axis_index("s") * 2 + lax.axis_index("c")
        base_w = wid * EPW
        pltpu.sync_copy(ia.at[pl.ds(base_w, EPW)], ia_v)
        pltpu.sync_copy(ib.at[pl.ds(base_w, EPW)], ib_v)

        def start(ci, slot):
            isl = pl.ds(ci * CHUNK, CHUNK)
            pltpu.async_copy(ta.at[ia_v.at[isl]], bufs_a[slot], g_sems[slot])
            pltpu.async_copy(tb.at[ib_v.at[isl]], bufs_b[slot], g_sems[slot])

        def drain_g(slot):
            isl = pl.ds(0, CHUNK)
            pltpu.make_async_copy(ta.at[ia_v.at[isl]], bufs_a[slot],
                                  g_sems[slot]).wait()
            pltpu.make_async_copy(tb.at[ib_v.at[isl]], bufs_b[slot],
                                  g_sems[slot]).wait()

        def drain_w(slot):
            pltpu.make_async_copy(bufs_o[slot], out.at[pl.ds(0, CHUNK)],
                                  w_sems[slot]).wait()

        def combine(slot):
            ba, bb, bo = bufs_a[slot], bufs_b[slot], bufs_o[slot]

            @plsc.parallel_loop(0, CHUNK, step=1, unroll=4)
            def _(r):
                for c in range(OW // 16):
                    sl = pl.ds(c * 16, 16)
                    a = ba[r, sl]
                    b = bb[r, sl]
                    if op == "sqdiff":
                        d = a - b
                        bo[r, sl] = d * d
                    elif op == "add":
                        bo[r, sl] = a + b
                    else:
                        bo[r, sl] = a * b

        def write(ci, slot):
            pltpu.async_copy(bufs_o[slot], out.at[pl.ds(base_w + ci * CHUNK,
                                                        CHUNK)], w_sems[slot])

        start(0, 0)

        def pair(k, carry):
            ci = 2 * k
            start(ci + 1, 1)
            drain_g(0)

            @pl.when(k > 0)
            def _():
                drain_w(0)

            combine(0)
            write(ci, 0)

            @pl.when(ci + 2 < NCHUNK)
            def _():
                start(ci + 2, 0)

            drain_g(1)

            @pl.when(k > 0)
            def _():
                drain_w(1)

            combine(1)
            write(ci + 1, 1)
            return carry

        lax.fori_loop(0, (NCHUNK - 1) // 2, pair, 0)
        drain_g(0)
        drain_w(0)
        combine(0)
        pltpu.sync_copy(bufs_o[0], out.at[pl.ds(base_w + (NCHUNK - 1) * CHUNK,
                                                CHUNK)])
        drain_w(1)

    fn = pl.kernel(
        body,
        out_type=jax.ShapeDtypeStruct((E, OW), jnp.float32),
        mesh=mesh,
        scratch_types=[
            pltpu.VMEM((EPW,), jnp.int32),
            pltpu.VMEM((EPW,), jnp.int32),
            [pltpu.VMEM((CHUNK, D), jnp.float32)] * 2,
            [pltpu.VMEM((CHUNK, D), jnp.float32)] * 2,
            [pltpu.VMEM((CHUNK, OW), jnp.float32)] * 2,
            [pltpu.SemaphoreType.DMA] * 2,
            [pltpu.SemaphoreType.DMA] * 2,
        ],
    )
    return fn(tab_a, tab_b, idx_a, idx_b)


NPAD = 10240


def _sc_scatter_add(m, idx, zeros_n):
    mesh = plsc.VectorSubcoreMesh(core_axis_name="c", subcore_axis_name="s",
                                  num_cores=2, num_subcores=16)
    rows = NPAD // 16

    def body(m_hbm, idx_hbm, z_hbm, out, idx_vs, bufs, agg_sh, sems):
        cid = lax.axis_index("c")
        sid = lax.axis_index("s")
        wid = sid * 2 + cid
        base_w = wid * EPW
        pltpu.sync_copy(z_hbm.at[pl.ds(sid * rows, rows)],
                        agg_sh.at[pl.ds(sid * rows, rows)])
        plsc.subcore_barrier()

        def start(ci, slot):
            base = base_w + ci * CHUNK
            pltpu.async_copy(idx_hbm.at[pl.ds(base, CHUNK)], idx_vs[slot],
                             sems[slot])
            pltpu.async_copy(m_hbm.at[pl.ds(base, CHUNK)], bufs[slot],
                             sems[slot])

        def drain(slot):
            pltpu.make_async_copy(idx_hbm.at[pl.ds(0, CHUNK)], idx_vs[slot],
                                  sems[slot]).wait()
            pltpu.make_async_copy(m_hbm.at[pl.ds(0, CHUNK)], bufs[slot],
                                  sems[slot]).wait()

        def scat(slot):
            pltpu.sync_copy(bufs[slot], agg_sh.at[idx_vs[slot]], add=True)

        start(0, 0)

        def pair(k, carry):
            ci = 2 * k
            start(ci + 1, 1)
            drain(0)
            scat(0)

            @pl.when(ci + 2 < NCHUNK)
            def _():
                start(ci + 2, 0)

            drain(1)
            scat(1)
            return carry

        lax.fori_loop(0, (NCHUNK - 1) // 2, pair, 0)
        drain(0)
        scat(0)
        plsc.subcore_barrier()
        pltpu.sync_copy(agg_sh.at[pl.ds(sid * rows, rows)],
                        out.at[cid, pl.ds(sid * rows, rows)])

    fn = pl.kernel(
        body,
        out_type=jax.ShapeDtypeStruct((2, NPAD, H), jnp.float32),
        mesh=mesh,
        scratch_types=[
            [pltpu.VMEM((CHUNK,), jnp.int32)] * 2,
            [pltpu.VMEM((CHUNK, H), jnp.float32)] * 2,
            pltpu.VMEM_SHARED((NPAD, H), jnp.float32),
            [pltpu.SemaphoreType.DMA] * 2,
        ],
    )
    return fn(m, idx, zeros_n)


def _rbf_feats(d2):
    d = jnp.sqrt(d2 + 1e-12)
    dn = d / CUTOFF
    inv = 1.0 / (dn + 1e-12)
    dn2 = dn * dn
    dn4 = dn2 * dn2
    dn5 = dn4 * dn
    env = (inv - 28.0 * dn5 + 48.0 * dn5 * dn - 21.0 * dn5 * dn2)
    env = env * (dn < 1.0).astype(jnp.float32)
    ki = lax.broadcasted_iota(jnp.int32, (1, 8), 1)
    arg = (ki.astype(jnp.float32) + 1.0) * jnp.pi * dn
    sn = jnp.sin(arg)
    mask = (ki < 6).astype(jnp.float32)
    return env * sn * mask


def _edge0_body(sq_ref, s_ref, rbfw_ref, wr_ref, wm_ref, e_ref, m_ref,
                rbfh_ref):
    d2 = jnp.sum(sq_ref[...], axis=1, keepdims=True)
    rbf = _rbf_feats(d2)
    rbfw = rbfw_ref[...]
    w6 = jnp.dot(rbfw, wr_ref[...], preferred_element_type=jnp.float32)
    pre = s_ref[...] + jnp.dot(rbf, w6, preferred_element_type=jnp.float32)
    e = _swish(pre)
    e_ref[...] = e
    rbf_h = jnp.dot(rbf, rbfw, preferred_element_type=jnp.float32)
    rbfh_ref[...] = rbf_h
    m_ref[...] = _swish(jnp.dot(e, wm_ref[...],
                                preferred_element_type=jnp.float32)) * rbf_h


def _tc_edge0(sq, s, rbf8, wr, wm):
    grid = (E // BE,)
    eh = pl.BlockSpec((BE, H), lambda b: (b, 0))
    return pl.pallas_call(
        _edge0_body,
        grid=grid,
        in_specs=[
            pl.BlockSpec((BE, 16), lambda b: (b, 0)),
            eh,
            pl.BlockSpec((8, H), lambda b: (0, 0)),
            pl.BlockSpec((H, H), lambda b: (0, 0)),
            pl.BlockSpec((H, H), lambda b: (0, 0)),
        ],
        out_specs=[eh, eh, eh],
        out_shape=[jax.ShapeDtypeStruct((E, H), jnp.float32)] * 3,
    )(sq, s, rbf8, wr, wm)


def _edge_body(rbfh_ref, e_ref, g_ref, wm_ref, *out_refs):
    e = e_ref[...] + _swish(g_ref[...])
    m = _swish(jnp.dot(e, wm_ref[...],
                       preferred_element_type=jnp.float32)) * rbfh_ref[...]
    if len(out_refs) == 2:
        out_refs[0][...] = e
        out_refs[1][...] = m
    else:
        out_refs[0][...] = m


def _tc_edge(rbfh, e, g, wm, want_e):
    grid = (E // BE,)
    eh = pl.BlockSpec((BE, H), lambda b: (b, 0))
    outs = [jax.ShapeDtypeStruct((E, H), jnp.float32)]
    out_specs = [eh]
    if want_e:
        outs = outs * 2
        out_specs = [eh, eh]
    res = pl.pallas_call(
        _edge_body,
        grid=grid,
        in_specs=[
            eh,
            eh,
            eh,
            pl.BlockSpec((H, H), lambda b: (0, 0)),
        ],
        out_specs=out_specs,
        out_shape=outs,
    )(rbfh, e, g, wm)
    return res if want_e else res[0]


def _init_body(z_ref, emb_ref, wj_ref, wi_ref, h0_ref, hjp_ref, hip_ref):
    z = z_ref[...]
    oh = (z == lax.broadcasted_iota(jnp.int32, (BN, 96), 1))
    h0 = jnp.dot(oh.astype(jnp.float32), emb_ref[...],
                 preferred_element_type=jnp.float32)
    h0_ref[...] = h0
    hjp_ref[...] = jnp.dot(h0, wj_ref[...], preferred_element_type=jnp.float32)
    hip_ref[...] = jnp.dot(h0, wi_ref[...], preferred_element_type=jnp.float32)


def _tc_init(z2, emb96, wj, wi):
    grid = (N // BN,)
    nh = pl.BlockSpec((BN, H), lambda b: (b, 0))
    return pl.pallas_call(
        _init_body,
        grid=grid,
        in_specs=[
            pl.BlockSpec((BN, 1), lambda b: (b, 0)),
            pl.BlockSpec((96, H), lambda b: (0, 0)),
            pl.BlockSpec((H, H), lambda b: (0, 0)),
            pl.BlockSpec((H, H), lambda b: (0, 0)),
        ],
        out_specs=[nh, nh, nh],
        out_shape=[jax.ShapeDtypeStruct((N, H), jnp.float32)] * 3,
    )(z2, emb96, wj, wi)


def _node_body(h_ref, agg_ref, w_ref, out_ref):
    a = agg_ref[0] + agg_ref[1]
    out_ref[...] = h_ref[...] + _swish(
        jnp.dot(a, w_ref[...], preferred_element_type=jnp.float32))


def _tc_node_update(h, agg2, w):
    grid = (N // BN,)
    nh = pl.BlockSpec((BN, H), lambda b: (b, 0))
    return pl.pallas_call(
        _node_body,
        grid=grid,
        in_specs=[
            nh,
            pl.BlockSpec((2, BN, H), lambda b: (0, b, 0)),
            pl.BlockSpec((H, H), lambda b: (0, 0)),
        ],
        out_specs=nh,
        out_shape=jax.ShapeDtypeStruct((N, H), jnp.float32),
    )(h, agg2, w)


def _out_body(h_ref, b_ref, o1_ref, o2_ref, o3_ref, w1_ref, b1_ref, w2_ref,
              b2_ref, pred_ref, ge_ref):
    k = pl.program_id(0)

    @pl.when(k == 0)
    def _():
        ge_ref[...] = jnp.zeros_like(ge_ref)

    hh = _swish(jnp.dot(h_ref[...], o1_ref[...],
                        preferred_element_type=jnp.float32))
    hh = _swish(jnp.dot(hh, o2_ref[...], preferred_element_type=jnp.float32))
    no = jnp.dot(hh, o3_ref[...], preferred_element_type=jnp.float32)
    oh = (b_ref[...] == lax.broadcasted_iota(jnp.int32, (BN, NG), 1))
    ge_ref[...] += lax.dot_general(
        oh.astype(jnp.float32), no, (((0,), (0,)), ((), ())),
        preferred_element_type=jnp.float32)

    @pl.when(k == (N // BN) - 1)
    def _():
        ge = ge_ref[...]
        hid = jnp.maximum(ge * w1_ref[...] + b1_ref[...], 0.0)
        pred_ref[...] = jnp.dot(hid, w2_ref[...],
                                preferred_element_type=jnp.float32) + b2_ref[...]


def _tc_output(h, batch2, o1, o2, o3, w1, b1, w2, b2):
    grid = (N // BN,)
    return pl.pallas_call(
        _out_body,
        grid=grid,
        in_specs=[
            pl.BlockSpec((BN, H), lambda b: (b, 0)),
            pl.BlockSpec((BN, 1), lambda b: (b, 0)),
            pl.BlockSpec((H, 256), lambda b: (0, 0)),
            pl.BlockSpec((256, 256), lambda b: (0, 0)),
            pl.BlockSpec((256, 1), lambda b: (0, 0)),
            pl.BlockSpec((1, 64), lambda b: (0, 0)),
            pl.BlockSpec((1, 64), lambda b: (0, 0)),
            pl.BlockSpec((64, 1), lambda b: (0, 0)),
            pl.BlockSpec((1, 1), lambda b: (0, 0)),
        ],
        out_specs=pl.BlockSpec((NG, 1), lambda b: (0, 0)),
        out_shape=jax.ShapeDtypeStruct((NG, 1), jnp.float32),
        scratch_shapes=[pltpu.VMEM((NG, 1), jnp.float32)],
    )(h, batch2, o1, o2, o3, w1, b1, w2, b2)


def kernel(x, pos, batch, edge_index, emb_z, lin_rbf, lin_edge, W_msg, W_upd,
           lin_out1, lin_out2, lin_out3, head_w1, head_b1, head_w2, head_b2):
    i = edge_index[0].astype(jnp.int32)
    j = edge_index[1].astype(jnp.int32)
    pos128 = jnp.pad(pos.astype(jnp.float32), ((0, 0), (0, H - 3)))
    emb96 = jnp.pad(emb_z, ((0, 1), (0, 0)))
    rbf8 = jnp.pad(lin_rbf, ((0, 2), (0, 0)))
    z2 = x.reshape(N, 1).astype(jnp.int32)
    batch2 = batch.reshape(N, 1).astype(jnp.int32)
    b1 = head_b1.reshape(1, 64)
    b2 = head_b2.reshape(1, 1)
    wj = lin_edge[0:H]
    wi = lin_edge[H:2 * H]
    wr = lin_edge[2 * H:3 * H]
    zeros_n = jnp.zeros((NPAD, H), jnp.float32)

    sq = _sc_gather_combine(pos128, pos128, i, j, "sqdiff", out_w=16)
    h, hjp, hip = _tc_init(z2, emb96, wj, wi)
    s = _sc_gather_combine(hjp, hip, j, i, "add")
    e, m, rbfh = _tc_edge0(sq, s, rbf8, wr, W_msg[0])
    for l in range(4):
        agg2 = _sc_scatter_add(m, i, zeros_n)
        h = _tc_node_update(h, agg2, W_upd[l])
        if l < 3:
            g = _sc_gather_combine(h, h, j, i, "mul")
            if l < 2:
                e, m = _tc_edge(rbfh, e, g, W_msg[l + 1], True)
            else:
                m = _tc_edge(rbfh, e, g, W_msg[l + 1], False)
    return _tc_output(h, batch2, lin_out1, lin_out2, lin_out3,
                      head_w1, b1, head_w2, b2)
